# baseline (device time: 78722 ns/iter reference)
import jax
import jax.numpy as jnp
from jax import lax
from jax.experimental import pallas as pl
from jax.experimental.pallas import tpu as pltpu

N_DEV = 8
N_TOK = 2048
D = 512
H = 1024
E_LOCAL = 4
CHUNK = N_TOK // N_DEV

ORDERS = ((1, 2, 4), (2, 4, 1), (4, 1, 2))
COLS = ((0, 384), (384, 384), (768, 256))
W_MAX = 384
RS_SLOT_BASE = (0, 4, 6)
AG_SLOT_BASE = (0, 1, 3)


def _subset_masks(masks):
    out = [0]
    for m in masks:
        out = out + [o | m for o in out]
    return out


def _lmap(v):
    return (v & 4) | ((v & 3) ^ ((v & 3) >> 1))


def kernel(x, router_W, route_idx, expert_W, shared_W):
    def body(x_ref, rw_ref, idx_ref, ew_ref, sw_ref, out_ref,
             acc_ref, wire_ref, xw_ref, recv_ref,
             rs_send_sem, rs_recv_sem, ag_send_sem, ag_recv_sem):
        p = lax.axis_index("i")
        b = _lmap(p)
        rows_b = pl.ds(b * CHUNK, CHUNK)

        xv = x_ref[:, :]
        scores = jnp.dot(xv, rw_ref[:, :], preferred_element_type=jnp.float32)
        s_max = jnp.max(scores, axis=-1, keepdims=True)
        e_s = jnp.exp(scores - s_max)
        probs = e_s / jnp.sum(e_s, axis=-1, keepdims=True)
        idx = idx_ref[:, :]
        onehot = (lax.broadcasted_iota(jnp.int32, scores.shape, 1) == idx)
        gate = jnp.sum(jnp.where(onehot, probs, 0.0), axis=-1, keepdims=True)

        xb = xv.astype(jnp.bfloat16)
        for k in range(E_LOCAL):
            e_id = p * E_LOCAL + k
            w_k = jnp.where(idx == e_id, gate, 0.0)
            xw_ref[k] = w_k.astype(jnp.bfloat16) * xb

        barrier_sem = pltpu.get_barrier_semaphore()
        for m in (1, 2, 4):
            pl.semaphore_signal(
                barrier_sem, inc=1,
                device_id=(_lmap(b ^ m),),
                device_id_type=pl.DeviceIdType.MESH,
            )
        pl.semaphore_wait(barrier_sem, 3)

        def compute_third(j):
            c0, w = COLS[j]
            cols = slice(c0, c0 + w)
            accj = jnp.zeros((N_TOK, w), jnp.float32)
            for k in range(E_LOCAL):
                accj = accj + jnp.dot(
                    xw_ref[k], ew_ref[k][:, cols].astype(jnp.bfloat16),
                    preferred_element_type=jnp.float32)
            acc_ref[:, cols] = accj

        def rs_send(j, s):
            m = ORDERS[j][s]
            done = sum(ORDERS[j][:s])
            free = ORDERS[j][s + 1:]
            partner = _lmap(b ^ m)
            c0, w = COLS[j]
            keep = done | m
            handles = []
            for ti, t in enumerate(_subset_masks(free)):
                slot = RS_SLOT_BASE[s] + ti
                c_send = ((b ^ m) & keep) | t
                rows_s = pl.ds(c_send * CHUNK, CHUNK)
                cols_s = slice(c0, c0 + w)
                wire_ref[rows_s, cols_s] = (
                    acc_ref[rows_s, cols_s].astype(jnp.bfloat16)
                )
                rdma = pltpu.make_async_remote_copy(
                    src_ref=wire_ref.at[rows_s, cols_s],
                    dst_ref=recv_ref.at[j, slot, :, pl.ds(0, w)],
                    send_sem=rs_send_sem.at[j, slot],
                    recv_sem=rs_recv_sem.at[j, slot],
                    device_id=(partner,),
                    device_id_type=pl.DeviceIdType.MESH,
                )
                rdma.start()
                c_recv = (b & keep) | t
                handles.append((rdma, slot, c_recv, c0, w))
            return handles

        def rs_finish(j, handles):
            for rdma, slot, c_recv, c0, w in handles:
                rdma.wait()
                rows = pl.ds(c_recv * CHUNK, CHUNK)
                cols = slice(c0, c0 + w)
                acc_ref[rows, cols] = (
                    acc_ref[rows, cols]
                    + recv_ref[j, slot, :, :w].astype(jnp.float32)
                )

        def shared_and_stage(j):
            c0, w = COLS[j]
            cols = slice(c0, c0 + w)
            wire_ref[rows_b, cols] = (
                acc_ref[rows_b, cols]
                + jnp.dot(x_ref[rows_b, :], sw_ref[:, cols],
                          preferred_element_type=jnp.float32)
            ).astype(jnp.bfloat16)
            out_ref[rows_b, cols] = wire_ref[rows_b, cols].astype(jnp.float32)

        def ag_send(j, s):
            rev = ORDERS[j][::-1]
            m = rev[s]
            partner = _lmap(b ^ m)
            c0, w = COLS[j]
            handles = []
            for ti, t in enumerate(_subset_masks(rev[:s])):
                slot = AG_SLOT_BASE[s] + ti
                c_send = b ^ t
                rows = pl.ds(c_send * CHUNK, CHUNK)
                cols = slice(c0, c0 + w)
                rdma = pltpu.make_async_remote_copy(
                    src_ref=wire_ref.at[rows, cols],
                    dst_ref=wire_ref.at[rows, cols],
                    send_sem=ag_send_sem.at[j, slot],
                    recv_sem=ag_recv_sem.at[j, slot],
                    device_id=(partner,),
                    device_id_type=pl.DeviceIdType.MESH,
                )
                rdma.start()
                c_recv = b ^ m ^ t
                handles.append((rdma, c_recv, c0, w))
            return handles

        def ag_finish(handles):
            for rdma, c_recv, c0, w in handles:
                rdma.wait()
                rows = pl.ds(c_recv * CHUNK, CHUNK)
                cols = slice(c0, c0 + w)
                out_ref[rows, cols] = wire_ref[rows, cols].astype(jnp.float32)

        rs_h = {}
        ag_h = {}
        for j in range(3):
            compute_third(j)
            rs_h[j] = rs_send(j, 0)
        for s in (0, 1):
            for j in range(3):
                rs_finish(j, rs_h[j])
                rs_h[j] = rs_send(j, s + 1)
        for j in range(3):
            rs_finish(j, rs_h[j])
            shared_and_stage(j)
            ag_h[j] = ag_send(j, 0)
        for s in (0, 1):
            for j in range(3):
                ag_finish(ag_h[j])
                ag_h[j] = ag_send(j, s + 1)
        for j in range(3):
            ag_finish(ag_h[j])

    return pl.pallas_call(
        body,
        out_shape=jax.ShapeDtypeStruct((N_TOK, H), jnp.float32),
        in_specs=[
            pl.BlockSpec(memory_space=pltpu.VMEM),
            pl.BlockSpec(memory_space=pltpu.VMEM),
            pl.BlockSpec(memory_space=pltpu.VMEM),
            pl.BlockSpec(memory_space=pltpu.VMEM),
            pl.BlockSpec(memory_space=pltpu.VMEM),
        ],
        out_specs=pl.BlockSpec(memory_space=pltpu.VMEM),
        scratch_shapes=[
            pltpu.VMEM((N_TOK, H), jnp.float32),
            pltpu.VMEM((N_TOK, H), jnp.bfloat16),
            pltpu.VMEM((E_LOCAL, N_TOK, D), jnp.bfloat16),
            pltpu.VMEM((3, 7, CHUNK, W_MAX), jnp.bfloat16),
            pltpu.SemaphoreType.DMA((3, 7)),
            pltpu.SemaphoreType.DMA((3, 7)),
            pltpu.SemaphoreType.DMA((3, 7)),
            pltpu.SemaphoreType.DMA((3, 7)),
        ],
        compiler_params=pltpu.CompilerParams(
            collective_id=0,
            vmem_limit_bytes=100 * 1024 * 1024,
        ),
    )(x, router_W, route_idx, expert_W, shared_W)


# device time: 74449 ns/iter; 1.0574x vs baseline; 1.0574x over previous
import jax
import jax.numpy as jnp
from jax import lax
from jax.experimental import pallas as pl
from jax.experimental.pallas import tpu as pltpu

N_DEV = 8
N_TOK = 2048
D = 512
H = 1024
E_LOCAL = 4
CHUNK = N_TOK // N_DEV

ORDERS = ((1, 2, 4), (2, 4, 1), (4, 1, 2))
COLS = ((0, 384), (384, 384), (768, 256))
W_MAX = 384
RS_SLOT_BASE = (0, 4, 6)
AG_SLOT_BASE = (0, 1, 3)


def _subset_masks(masks):
    out = [0]
    for m in masks:
        out = out + [o | m for o in out]
    return out


def _lmap(v):
    return (v & 4) | ((v & 3) ^ ((v & 3) >> 1))


def kernel(x, router_W, route_idx, expert_W, shared_W):
    def body(x_ref, rw_ref, idx_ref, ew_ref, sw_ref, out_ref,
             acc_ref, xw_ref, recv_ref,
             rs_send_sem, rs_recv_sem, ag_send_sem, ag_recv_sem):
        p = lax.axis_index("i")
        b = _lmap(p)
        rows_b = pl.ds(b * CHUNK, CHUNK)

        xv = x_ref[:, :]
        scores = jnp.dot(xv, rw_ref[:, :], preferred_element_type=jnp.float32)
        s_max = jnp.max(scores, axis=-1, keepdims=True)
        e_s = jnp.exp(scores - s_max)
        probs = e_s / jnp.sum(e_s, axis=-1, keepdims=True)
        idx = idx_ref[:, :]
        onehot = (lax.broadcasted_iota(jnp.int32, scores.shape, 1) == idx)
        gate = jnp.sum(jnp.where(onehot, probs, 0.0), axis=-1, keepdims=True)

        xb = xv.astype(jnp.bfloat16)
        for k in range(E_LOCAL):
            e_id = p * E_LOCAL + k
            w_k = jnp.where(idx == e_id, gate, 0.0)
            xw_ref[k] = w_k.astype(jnp.bfloat16) * xb

        barrier_sem = pltpu.get_barrier_semaphore()
        for m in (1, 2, 4):
            pl.semaphore_signal(
                barrier_sem, inc=1,
                device_id=(_lmap(b ^ m),),
                device_id_type=pl.DeviceIdType.MESH,
            )
        pl.semaphore_wait(barrier_sem, 3)

        def compute_third(j):
            c0, w = COLS[j]
            cols = slice(c0, c0 + w)
            accj = jnp.zeros((N_TOK, w), jnp.float32)
            for k in range(E_LOCAL):
                accj = accj + jnp.dot(
                    xw_ref[k], ew_ref[k][:, cols].astype(jnp.bfloat16),
                    preferred_element_type=jnp.float32)
            acc_ref[:, cols] = accj.astype(jnp.bfloat16)

        def rs_send(j, s):
            m = ORDERS[j][s]
            done = sum(ORDERS[j][:s])
            free = ORDERS[j][s + 1:]
            partner = _lmap(b ^ m)
            c0, w = COLS[j]
            keep = done | m
            handles = []
            for ti, t in enumerate(_subset_masks(free)):
                slot = RS_SLOT_BASE[s] + ti
                c_send = ((b ^ m) & keep) | t
                rows_s = pl.ds(c_send * CHUNK, CHUNK)
                cols_s = slice(c0, c0 + w)
                rdma = pltpu.make_async_remote_copy(
                    src_ref=acc_ref.at[rows_s, cols_s],
                    dst_ref=recv_ref.at[j, slot, :, pl.ds(0, w)],
                    send_sem=rs_send_sem.at[j, slot],
                    recv_sem=rs_recv_sem.at[j, slot],
                    device_id=(partner,),
                    device_id_type=pl.DeviceIdType.MESH,
                )
                rdma.start()
                c_recv = (b & keep) | t
                handles.append((rdma, slot, c_recv, c0, w))
            return handles

        def rs_finish(j, handles):
            for rdma, slot, c_recv, c0, w in handles:
                rdma.wait()
                rows = pl.ds(c_recv * CHUNK, CHUNK)
                cols = slice(c0, c0 + w)
                acc_ref[rows, cols] = (
                    acc_ref[rows, cols] + recv_ref[j, slot, :, :w]
                )

        def shared_and_stage(j, sh):
            c0, w = COLS[j]
            cols = slice(c0, c0 + w)
            out_ref[rows_b, cols] = (
                acc_ref[rows_b, cols].astype(jnp.float32) + sh[:, cols]
            ).astype(jnp.bfloat16)

        def ag_send(j, s):
            rev = ORDERS[j][::-1]
            m = rev[s]
            partner = _lmap(b ^ m)
            c0, w = COLS[j]
            handles = []
            for ti, t in enumerate(_subset_masks(rev[:s])):
                slot = AG_SLOT_BASE[s] + ti
                c_send = b ^ t
                rows = pl.ds(c_send * CHUNK, CHUNK)
                cols = slice(c0, c0 + w)
                rdma = pltpu.make_async_remote_copy(
                    src_ref=out_ref.at[rows, cols],
                    dst_ref=out_ref.at[rows, cols],
                    send_sem=ag_send_sem.at[j, slot],
                    recv_sem=ag_recv_sem.at[j, slot],
                    device_id=(partner,),
                    device_id_type=pl.DeviceIdType.MESH,
                )
                rdma.start()
                c_recv = b ^ m ^ t
                handles.append((rdma, c_recv, c0, w))
            return handles

        def ag_finish(handles):
            for rdma, c_recv, c0, w in handles:
                rdma.wait()

        rs_h = {}
        ag_h = {}
        for j in range(3):
            compute_third(j)
            rs_h[j] = rs_send(j, 0)
        for j in range(3):
            rs_finish(j, rs_h[j])
            rs_h[j] = rs_send(j, 1)
        sh = jnp.dot(x_ref[rows_b, :], sw_ref[:, :],
                     preferred_element_type=jnp.float32)
        for j in range(3):
            rs_finish(j, rs_h[j])
            rs_h[j] = rs_send(j, 2)
        for j in range(3):
            rs_finish(j, rs_h[j])
            shared_and_stage(j, sh)
            ag_h[j] = ag_send(j, 0)
        for s in (0, 1):
            for j in range(3):
                ag_finish(ag_h[j])
                ag_h[j] = ag_send(j, s + 1)
        for j in range(3):
            ag_finish(ag_h[j])

    return pl.pallas_call(
        body,
        out_shape=jax.ShapeDtypeStruct((N_TOK, H), jnp.bfloat16),
        in_specs=[
            pl.BlockSpec(memory_space=pltpu.VMEM),
            pl.BlockSpec(memory_space=pltpu.VMEM),
            pl.BlockSpec(memory_space=pltpu.VMEM),
            pl.BlockSpec(memory_space=pltpu.VMEM),
            pl.BlockSpec(memory_space=pltpu.VMEM),
        ],
        out_specs=pl.BlockSpec(memory_space=pltpu.VMEM),
        scratch_shapes=[
            pltpu.VMEM((N_TOK, H), jnp.bfloat16),
            pltpu.VMEM((E_LOCAL, N_TOK, D), jnp.bfloat16),
            pltpu.VMEM((3, 7, CHUNK, W_MAX), jnp.bfloat16),
            pltpu.SemaphoreType.DMA((3, 7)),
            pltpu.SemaphoreType.DMA((3, 7)),
            pltpu.SemaphoreType.DMA((3, 7)),
            pltpu.SemaphoreType.DMA((3, 7)),
        ],
        compiler_params=pltpu.CompilerParams(
            collective_id=0,
            vmem_limit_bytes=100 * 1024 * 1024,
        ),
    )(x, router_W, route_idx, expert_W, shared_W)


# device time: 72268 ns/iter; 1.0893x vs baseline; 1.0302x over previous
import jax
import jax.numpy as jnp
from jax import lax
from jax.experimental import pallas as pl
from jax.experimental.pallas import tpu as pltpu

N_DEV = 8
N_TOK = 2048
D = 512
H = 1024
E_LOCAL = 4
CHUNK = N_TOK // N_DEV
HALF = CHUNK // 2

ORDERS = ((1, 2, 4), (2, 4, 1), (4, 1, 2))
COLS = ((0, 384), (384, 384), (768, 256))
W_MAX = 384
LANES = tuple((j, h) for j in range(3) for h in range(2))
RS_SLOT_BASE = (0, 4, 6)
AG_SLOT_BASE = (0, 1, 3)


def _subset_masks(masks):
    out = [0]
    for m in masks:
        out = out + [o | m for o in out]
    return out


def _lmap(v):
    return (v & 4) | ((v & 3) ^ ((v & 3) >> 1))


def kernel(x, router_W, route_idx, expert_W, shared_W):
    def body(x_ref, rw_ref, idx_ref, ew_ref, sw_ref, out_ref,
             acc_ref, xw_ref, recv_ref,
             rs_send_sem, rs_recv_sem, ag_send_sem, ag_recv_sem):
        p = lax.axis_index("i")
        b = _lmap(p)

        xv = x_ref[:, :]
        scores = jnp.dot(xv, rw_ref[:, :], preferred_element_type=jnp.float32)
        s_max = jnp.max(scores, axis=-1, keepdims=True)
        e_s = jnp.exp(scores - s_max)
        probs = e_s / jnp.sum(e_s, axis=-1, keepdims=True)
        idx = idx_ref[:, :]
        onehot = (lax.broadcasted_iota(jnp.int32, scores.shape, 1) == idx)
        gate = jnp.sum(jnp.where(onehot, probs, 0.0), axis=-1, keepdims=True)

        xb = xv.astype(jnp.bfloat16)
        for k in range(E_LOCAL):
            e_id = p * E_LOCAL + k
            w_k = jnp.where(idx == e_id, gate, 0.0)
            xw_ref[k] = w_k.astype(jnp.bfloat16) * xb

        barrier_sem = pltpu.get_barrier_semaphore()
        for m in (1, 2, 4):
            pl.semaphore_signal(
                barrier_sem, inc=1,
                device_id=(_lmap(b ^ m),),
                device_id_type=pl.DeviceIdType.MESH,
            )
        pl.semaphore_wait(barrier_sem, 3)

        def hrows(c, h):
            return pl.ds(c * CHUNK + h * HALF, HALF)

        def compute_third(j):
            c0, w = COLS[j]
            cols = slice(c0, c0 + w)
            accj = jnp.zeros((N_TOK, w), jnp.float32)
            for k in range(E_LOCAL):
                accj = accj + jnp.dot(
                    xw_ref[k], ew_ref[k][:, cols].astype(jnp.bfloat16),
                    preferred_element_type=jnp.float32)
            acc_ref[:, cols] = accj.astype(jnp.bfloat16)

        def rs_send(j, h, s):
            lane = j * 2 + h
            m = ORDERS[j][s]
            done = sum(ORDERS[j][:s])
            free = ORDERS[j][s + 1:]
            partner = _lmap(b ^ m)
            c0, w = COLS[j]
            keep = done | m
            handles = []
            for ti, t in enumerate(_subset_masks(free)):
                slot = RS_SLOT_BASE[s] + ti
                c_send = ((b ^ m) & keep) | t
                rdma = pltpu.make_async_remote_copy(
                    src_ref=acc_ref.at[hrows(c_send, h), slice(c0, c0 + w)],
                    dst_ref=recv_ref.at[lane, slot, :, pl.ds(0, w)],
                    send_sem=rs_send_sem.at[lane, slot],
                    recv_sem=rs_recv_sem.at[lane, slot],
                    device_id=(partner,),
                    device_id_type=pl.DeviceIdType.MESH,
                )
                rdma.start()
                c_recv = (b & keep) | t
                handles.append((rdma, slot, c_recv))
            return handles

        def rs_finish(j, h, handles):
            lane = j * 2 + h
            c0, w = COLS[j]
            cols = slice(c0, c0 + w)
            for rdma, slot, c_recv in handles:
                rdma.wait()
                rows = hrows(c_recv, h)
                acc_ref[rows, cols] = (
                    acc_ref[rows, cols] + recv_ref[lane, slot, :, :w]
                )

        def shared_and_stage(j, h, sh):
            c0, w = COLS[j]
            cols = slice(c0, c0 + w)
            rows = hrows(b, h)
            out_ref[rows, cols] = (
                acc_ref[rows, cols].astype(jnp.float32)
                + sh[h * HALF:(h + 1) * HALF, cols]
            ).astype(jnp.bfloat16)

        def ag_send(j, h, s):
            lane = j * 2 + h
            rev = ORDERS[j][::-1]
            m = rev[s]
            partner = _lmap(b ^ m)
            c0, w = COLS[j]
            cols = slice(c0, c0 + w)
            handles = []
            for ti, t in enumerate(_subset_masks(rev[:s])):
                slot = AG_SLOT_BASE[s] + ti
                c_send = b ^ t
                rows = hrows(c_send, h)
                rdma = pltpu.make_async_remote_copy(
                    src_ref=out_ref.at[rows, cols],
                    dst_ref=out_ref.at[rows, cols],
                    send_sem=ag_send_sem.at[lane, slot],
                    recv_sem=ag_recv_sem.at[lane, slot],
                    device_id=(partner,),
                    device_id_type=pl.DeviceIdType.MESH,
                )
                rdma.start()
                handles.append(rdma)
            return handles

        def ag_finish(handles):
            for rdma in handles:
                rdma.wait()

        rs_h = {}
        ag_h = {}
        for j in range(3):
            compute_third(j)
            for h in (0, 1):
                rs_h[(j, h)] = rs_send(j, h, 0)
        for j, h in LANES:
            rs_finish(j, h, rs_h[(j, h)])
            rs_h[(j, h)] = rs_send(j, h, 1)
        sh = jnp.dot(x_ref[pl.ds(b * CHUNK, CHUNK), :], sw_ref[:, :],
                     preferred_element_type=jnp.float32)
        for j, h in LANES:
            rs_finish(j, h, rs_h[(j, h)])
            rs_h[(j, h)] = rs_send(j, h, 2)
        for j, h in LANES:
            rs_finish(j, h, rs_h[(j, h)])
            shared_and_stage(j, h, sh)
            ag_h[(j, h)] = ag_send(j, h, 0)
        for s in (0, 1):
            for j, h in LANES:
                ag_finish(ag_h[(j, h)])
                ag_h[(j, h)] = ag_send(j, h, s + 1)
        for j, h in LANES:
            ag_finish(ag_h[(j, h)])

    return pl.pallas_call(
        body,
        out_shape=jax.ShapeDtypeStruct((N_TOK, H), jnp.bfloat16),
        in_specs=[
            pl.BlockSpec(memory_space=pltpu.VMEM),
            pl.BlockSpec(memory_space=pltpu.VMEM),
            pl.BlockSpec(memory_space=pltpu.VMEM),
            pl.BlockSpec(memory_space=pltpu.VMEM),
            pl.BlockSpec(memory_space=pltpu.VMEM),
        ],
        out_specs=pl.BlockSpec(memory_space=pltpu.VMEM),
        scratch_shapes=[
            pltpu.VMEM((N_TOK, H), jnp.bfloat16),
            pltpu.VMEM((E_LOCAL, N_TOK, D), jnp.bfloat16),
            pltpu.VMEM((6, 7, HALF, W_MAX), jnp.bfloat16),
            pltpu.SemaphoreType.DMA((6, 7)),
            pltpu.SemaphoreType.DMA((6, 7)),
            pltpu.SemaphoreType.DMA((6, 7)),
            pltpu.SemaphoreType.DMA((6, 7)),
        ],
        compiler_params=pltpu.CompilerParams(
            collective_id=0,
            vmem_limit_bytes=100 * 1024 * 1024,
        ),
    )(x, router_W, route_idx, expert_W, shared_W)


# device time: 70743 ns/iter; 1.1128x vs baseline; 1.0216x over previous
import jax
import jax.numpy as jnp
from jax import lax
from jax.experimental import pallas as pl
from jax.experimental.pallas import tpu as pltpu

N_DEV = 8
N_TOK = 2048
D = 512
H = 1024
E_LOCAL = 4
CHUNK = N_TOK // N_DEV
HALF = CHUNK // 4

ORDERS = ((1, 2, 4), (2, 4, 1), (4, 1, 2))
COLS = ((0, 384), (384, 384), (768, 256))
W_MAX = 384
LANES = tuple((j, h) for h in range(4) for j in range(3))
RS_SLOT_BASE = (0, 4, 6)
AG_SLOT_BASE = (0, 1, 3)


def _subset_masks(masks):
    out = [0]
    for m in masks:
        out = out + [o | m for o in out]
    return out


def _lmap(v):
    return (v & 4) | ((v & 3) ^ ((v & 3) >> 1))


def kernel(x, router_W, route_idx, expert_W, shared_W):
    def body(x_ref, rw_ref, idx_ref, ew_ref, sw_ref, out_ref,
             acc_ref, xw_ref, recv_ref,
             rs_send_sem, rs_recv_sem, ag_send_sem, ag_recv_sem):
        p = lax.axis_index("i")
        b = _lmap(p)

        xv = x_ref[:, :]
        scores = jnp.dot(xv, rw_ref[:, :], preferred_element_type=jnp.float32)
        s_max = jnp.max(scores, axis=-1, keepdims=True)
        e_s = jnp.exp(scores - s_max)
        probs = e_s / jnp.sum(e_s, axis=-1, keepdims=True)
        idx = idx_ref[:, :]
        onehot = (lax.broadcasted_iota(jnp.int32, scores.shape, 1) == idx)
        gate = jnp.sum(jnp.where(onehot, probs, 0.0), axis=-1, keepdims=True)

        xb = xv.astype(jnp.bfloat16)
        for k in range(E_LOCAL):
            e_id = p * E_LOCAL + k
            w_k = jnp.where(idx == e_id, gate, 0.0)
            xw_ref[k] = w_k.astype(jnp.bfloat16) * xb

        barrier_sem = pltpu.get_barrier_semaphore()
        for m in (1, 2, 4):
            pl.semaphore_signal(
                barrier_sem, inc=1,
                device_id=(_lmap(b ^ m),),
                device_id_type=pl.DeviceIdType.MESH,
            )
        pl.semaphore_wait(barrier_sem, 3)

        def hrows(c, h):
            return pl.ds(c * CHUNK + h * HALF, HALF)

        def compute_third(j):
            c0, w = COLS[j]
            cols = slice(c0, c0 + w)
            accj = jnp.zeros((N_TOK, w), jnp.float32)
            for k in range(E_LOCAL):
                accj = accj + jnp.dot(
                    xw_ref[k], ew_ref[k][:, cols].astype(jnp.bfloat16),
                    preferred_element_type=jnp.float32)
            acc_ref[:, cols] = accj.astype(jnp.bfloat16)

        def rs_send(j, h, s):
            lane = j * 4 + h
            m = ORDERS[j][s]
            done = sum(ORDERS[j][:s])
            free = ORDERS[j][s + 1:]
            partner = _lmap(b ^ m)
            c0, w = COLS[j]
            keep = done | m
            handles = []
            for ti, t in enumerate(_subset_masks(free)):
                slot = RS_SLOT_BASE[s] + ti
                c_send = ((b ^ m) & keep) | t
                rdma = pltpu.make_async_remote_copy(
                    src_ref=acc_ref.at[hrows(c_send, h), slice(c0, c0 + w)],
                    dst_ref=recv_ref.at[lane, slot, :, pl.ds(0, w)],
                    send_sem=rs_send_sem.at[lane, slot],
                    recv_sem=rs_recv_sem.at[lane, slot],
                    device_id=(partner,),
                    device_id_type=pl.DeviceIdType.MESH,
                )
                rdma.start()
                c_recv = (b & keep) | t
                handles.append((rdma, slot, c_recv))
            return handles

        def rs_finish(j, h, handles):
            lane = j * 4 + h
            c0, w = COLS[j]
            cols = slice(c0, c0 + w)
            for rdma, slot, c_recv in handles:
                rdma.wait()
                rows = hrows(c_recv, h)
                acc_ref[rows, cols] = (
                    acc_ref[rows, cols] + recv_ref[lane, slot, :, :w]
                )

        def shared_and_stage(j, h, sh):
            c0, w = COLS[j]
            cols = slice(c0, c0 + w)
            rows = hrows(b, h)
            out_ref[rows, cols] = (
                acc_ref[rows, cols].astype(jnp.float32)
                + sh[h * HALF:(h + 1) * HALF, cols]
            ).astype(jnp.bfloat16)

        def ag_send(j, h, s):
            lane = j * 4 + h
            rev = ORDERS[j][::-1]
            m = rev[s]
            partner = _lmap(b ^ m)
            c0, w = COLS[j]
            cols = slice(c0, c0 + w)
            handles = []
            for ti, t in enumerate(_subset_masks(rev[:s])):
                slot = AG_SLOT_BASE[s] + ti
                c_send = b ^ t
                rows = hrows(c_send, h)
                rdma = pltpu.make_async_remote_copy(
                    src_ref=out_ref.at[rows, cols],
                    dst_ref=out_ref.at[rows, cols],
                    send_sem=ag_send_sem.at[lane, slot],
                    recv_sem=ag_recv_sem.at[lane, slot],
                    device_id=(partner,),
                    device_id_type=pl.DeviceIdType.MESH,
                )
                rdma.start()
                handles.append(rdma)
            return handles

        def ag_finish(handles):
            for rdma in handles:
                rdma.wait()

        rs_h = {}
        ag_h = {}
        for j in range(3):
            compute_third(j)
            for h in range(4):
                rs_h[(j, h)] = rs_send(j, h, 0)
        for j, h in LANES:
            rs_finish(j, h, rs_h[(j, h)])
            rs_h[(j, h)] = rs_send(j, h, 1)
        sh = jnp.dot(x_ref[pl.ds(b * CHUNK, CHUNK), :], sw_ref[:, :],
                     preferred_element_type=jnp.float32)
        for j, h in LANES:
            rs_finish(j, h, rs_h[(j, h)])
            rs_h[(j, h)] = rs_send(j, h, 2)
        for j, h in LANES:
            rs_finish(j, h, rs_h[(j, h)])
            shared_and_stage(j, h, sh)
            ag_h[(j, h)] = ag_send(j, h, 0)
        for s in (0, 1):
            for j, h in LANES:
                ag_finish(ag_h[(j, h)])
                ag_h[(j, h)] = ag_send(j, h, s + 1)
        for j, h in LANES:
            ag_finish(ag_h[(j, h)])

    return pl.pallas_call(
        body,
        out_shape=jax.ShapeDtypeStruct((N_TOK, H), jnp.bfloat16),
        in_specs=[
            pl.BlockSpec(memory_space=pltpu.VMEM),
            pl.BlockSpec(memory_space=pltpu.VMEM),
            pl.BlockSpec(memory_space=pltpu.VMEM),
            pl.BlockSpec(memory_space=pltpu.VMEM),
            pl.BlockSpec(memory_space=pltpu.VMEM),
        ],
        out_specs=pl.BlockSpec(memory_space=pltpu.VMEM),
        scratch_shapes=[
            pltpu.VMEM((N_TOK, H), jnp.bfloat16),
            pltpu.VMEM((E_LOCAL, N_TOK, D), jnp.bfloat16),
            pltpu.VMEM((12, 7, HALF, W_MAX), jnp.bfloat16),
            pltpu.SemaphoreType.DMA((12, 7)),
            pltpu.SemaphoreType.DMA((12, 7)),
            pltpu.SemaphoreType.DMA((12, 7)),
            pltpu.SemaphoreType.DMA((12, 7)),
        ],
        compiler_params=pltpu.CompilerParams(
            collective_id=0,
            vmem_limit_bytes=100 * 1024 * 1024,
        ),
    )(x, router_W, route_idx, expert_W, shared_W)
